# Initial kernel scaffold; baseline (speedup 1.0000x reference)
import jax
import jax.numpy as jnp
from jax.experimental import pallas as pl


def kernel(xyzs, imu_data, params):
    raise NotImplementedError("write your pallas kernel here")


# speedup vs baseline: 18.0761x; 18.0761x over previous
import jax
import jax.numpy as jnp
from jax.experimental import pallas as pl

_NS = 9.0

_CFGS = [
    {"r": 0.1, "tk": 1, "ss": 2, "ts": 1, "pad": (0, 0), "inp": 0,    "mid": 45,   "out": 64},
    {"r": 0.2, "tk": 3, "ss": 2, "ts": 2, "pad": (1, 2), "inp": 64,   "mid": 96,   "out": 128},
    {"r": 0.2, "tk": 3, "ss": 1, "ts": 1, "pad": (1, 2), "inp": 128,  "mid": 192,  "out": 256},
    {"r": 0.4, "tk": 3, "ss": 2, "ts": 2, "pad": (1, 2), "inp": 256,  "mid": 384,  "out": 512},
    {"r": 0.4, "tk": 3, "ss": 1, "ts": 1, "pad": (1, 2), "inp": 512,  "mid": 768,  "out": 1024},
    {"r": 0.4, "tk": 1, "ss": 2, "ts": 1, "pad": (0, 0), "inp": 1024, "mid": 1536, "out": 2048},
]


def _plan(cfg, F):
    pad0, pad1 = cfg["pad"]
    trad = (cfg["tk"] - 1) // 2
    Fp = F + pad0 + pad1
    t_list = list(range(trad, Fp - trad, cfg["ts"]))

    def orig(p):
        return min(max(p - pad0, 0), F - 1)

    centers = [orig(t) for t in t_list]
    nbrs = [[orig(p) for p in range(t - trad, t + trad + 1)] for t in t_list]
    return centers, nbrs


def _fps(pts, M):
    G, _, N = pts.shape
    niota = jax.lax.broadcasted_iota(jnp.int32, (G, N), 1)
    miota = jax.lax.broadcasted_iota(jnp.int32, (1, 1, M), 2)
    p0 = pts[:, :, 0:1]
    anch0 = jnp.where(miota == 0, p0, 0.0)
    d0 = jnp.full((G, N), 1e10, dtype=jnp.float32)

    def body(i, carry):
        d, last, anch = carry
        dist = ((pts[:, 0, :] - last[:, 0]) ** 2
                + (pts[:, 1, :] - last[:, 1]) ** 2
                + (pts[:, 2, :] - last[:, 2]) ** 2)
        d = jnp.minimum(d, dist)
        mx = jnp.max(d, axis=1, keepdims=True)
        nxt = jnp.min(jnp.where(d == mx, niota, N), axis=1, keepdims=True)
        oh = (niota == nxt).astype(jnp.float32)
        nl = jnp.concatenate(
            [jnp.sum(oh * pts[:, c, :], axis=1, keepdims=True) for c in range(3)],
            axis=1)[:, :, None]
        anch = anch + jnp.where(miota == i, 1.0, 0.0) * nl
        return d, nl, anch

    _, _, anch = jax.lax.fori_loop(1, M, body, (d0, p0, anch0))
    return anch


def _pair_feat(a_cm, x_cm, f_rm, wcatT, tri, r):
    B, _, M = a_cm.shape
    N = x_cm.shape[2]
    d2 = ((a_cm[:, 0, :, None] - x_cm[:, 0, None, :]) ** 2
          + (a_cm[:, 1, :, None] - x_cm[:, 1, None, :]) ** 2
          + (a_cm[:, 2, :, None] - x_cm[:, 2, None, :]) ** 2)
    within = d2 < r * r
    wf = within.astype(jnp.float32)
    cum = jax.lax.dot(wf.reshape(B * M, N), tri).reshape(B, M, N)
    sel = wf * (cum <= _NS).astype(jnp.float32)
    cnt = jnp.sum(wf, axis=2)
    extra = _NS - jnp.minimum(cnt, _NS)
    bi = jax.lax.broadcasted_iota(jnp.int32, (B, M, N), 2)
    first = jnp.min(jnp.where(within, bi, N), axis=2)
    first = jnp.where(first >= N, 0, first)
    oh = (bi == first[:, :, None]).astype(jnp.float32)
    w = sel + extra[:, :, None] * oh
    sx = jnp.concatenate(
        [jnp.sum(w * (x_cm[:, c, None, :] - a_cm[:, c, :, None]),
                 axis=2, keepdims=True) for c in range(3)],
        axis=2)
    if f_rm is not None:
        sf = jax.lax.dot_general(w, f_rm, (((2,), (1,)), ((0,), (0,))),
                                 precision=jax.lax.Precision.HIGHEST)
        s = jnp.concatenate([sx, sf], axis=2)
    else:
        s = sx
    pf = jax.lax.dot(s.reshape(B * M, s.shape[2]), wcatT,
                     precision=jax.lax.Precision.HIGHEST)
    return pf.reshape(B, M, wcatT.shape[1])


def _make_layer_kernel(cfg, F, N, B, out_relu):
    centers, nbrs = _plan(cfg, F)
    T = len(centers)
    M = N // cfg["ss"]
    inp, mid, out, tk, r, ss = (cfg["inp"], cfg["mid"], cfg["out"],
                                cfg["tk"], cfg["r"], cfg["ss"])
    ucf = sorted(set(centers))
    upairs = sorted({(centers[ti], nb) for ti in range(T) for nb in nbrs[ti]})

    def kfn(*refs):
        if inp:
            x_ref, f_ref, wcatT_ref, gamma_ref, beta_ref, wtT_ref, anch_ref, feat_ref = refs
        else:
            x_ref, wcatT_ref, gamma_ref, beta_ref, wtT_ref, anch_ref, feat_ref = refs
            f_ref = None
        X = [x_ref[c] for c in range(F)]
        wcatT = wcatT_ref[...]
        gamma = gamma_ref[...]
        beta = beta_ref[...]
        wtT = wtT_ref[...]

        if ss > 1:
            pts = jnp.concatenate([X[c] for c in ucf], axis=0)
            aa = _fps(pts, M)
            anch_cm = {c: aa[ui * B:(ui + 1) * B] for ui, c in enumerate(ucf)}
        else:
            anch_cm = {c: X[c] for c in ucf}

        ti_ = jax.lax.broadcasted_iota(jnp.int32, (N, N), 0)
        tj_ = jax.lax.broadcasted_iota(jnp.int32, (N, N), 1)
        tri = (ti_ <= tj_).astype(jnp.float32)

        pfd = {}
        for (c, nb) in upairs:
            fr = f_ref[nb] if f_ref is not None else None
            pfd[(c, nb)] = _pair_feat(anch_cm[c], X[nb], fr, wcatT, tri, r)

        for ti in range(T):
            c = centers[ti]
            h = jnp.concatenate([pfd[(c, nb)] for nb in nbrs[ti]], axis=2)
            mx = jnp.max(h, axis=(0, 1), keepdims=True)
            mn = jnp.min(h, axis=(0, 1), keepdims=True)
            mean = jnp.mean(h, axis=(0, 1), keepdims=True)
            mean = jnp.where(mx == mn, mx, mean)
            delta = h - mean
            var = jnp.mean(delta * delta, axis=(0, 1), keepdims=True)
            hn = delta / jnp.sqrt(var + 1e-5)
            hn = hn * gamma + beta
            hn = jnp.maximum(hn, 0.0)
            o = jax.lax.dot(hn.reshape(B * M, tk * mid), wtT,
                            precision=jax.lax.Precision.HIGHEST).reshape(B, M, out)
            if out_relu:
                o = jnp.maximum(o, 0.0)
            feat_ref[ti] = o
            anch_ref[ti] = anch_cm[c]

    return kfn, T, M


def _head_kernel(feat_ref, imu_ref, w1T_ref, b1_ref, w2T_ref, b2_ref,
                 w3T_ref, b3_ref, out_ref):
    fe = feat_ref[...]
    pooled = jnp.mean(fe, axis=2)
    feat = jnp.max(pooled, axis=0)
    x = jnp.concatenate([feat, imu_ref[...]], axis=1)
    hi = jax.lax.Precision.HIGHEST
    h = jnp.maximum(jax.lax.dot(x, w1T_ref[...], precision=hi) + b1_ref[...], 0.0)
    h = jnp.maximum(jax.lax.dot(h, w2T_ref[...], precision=hi) + b2_ref[...], 0.0)
    out_ref[...] = jax.lax.dot(h, w3T_ref[...], precision=hi) + b3_ref[...]


def kernel(xyzs, imu_data, params):
    B, K, _, N = xyzs.shape
    X = jnp.transpose(xyzs, (1, 0, 2, 3)).astype(jnp.float32)
    FT = None
    F = K
    for li, cfg in enumerate(_CFGS):
        p = params["layers"][li]
        kfn, T, M = _make_layer_kernel(cfg, F, N, B, out_relu=(li < len(_CFGS) - 1))
        if cfg["inp"]:
            wcatT = jnp.concatenate([p["Wd"], p["Wf"]], axis=1).T
        else:
            wcatT = p["Wd"].T
        gamma = p["gamma"].reshape(1, 1, -1)
        beta = p["beta"].reshape(1, 1, -1)
        wtT = p["Wt"].T
        out_shape = (
            jax.ShapeDtypeStruct((T, B, 3, M), jnp.float32),
            jax.ShapeDtypeStruct((T, B, M, cfg["out"]), jnp.float32),
        )
        args = (X,) + ((FT,) if cfg["inp"] else ()) + (wcatT, gamma, beta, wtT)
        X, FT = pl.pallas_call(kfn, out_shape=out_shape)(*args)
        F, N = T, M
    imu = imu_data.reshape(B, -1).astype(jnp.float32)
    out = pl.pallas_call(
        _head_kernel,
        out_shape=jax.ShapeDtypeStruct((B, 5), jnp.float32),
    )(FT, imu, params["W1"].T, params["b1"].reshape(1, -1),
      params["W2"].T, params["b2"].reshape(1, -1),
      params["W3"].T, params["b3"].reshape(1, -1))
    return out
